# baseline (speedup 1.0000x reference)
import functools

import jax
import jax.numpy as jnp
from jax import lax
from jax.experimental import pallas as pl
from jax.experimental.pallas import tpu as pltpu
from jax.experimental.pallas import tpu_sc as plsc

_N = 10000
_E = 320000
_HID = 64
_NUM_GRAPHS = 64
_EBLK = 6400

_NC = 2
_NS = 16
_NW = _NC * _NS
_C = 40
_EPT = _E // _NW
_NCH = _EPT // _C
_ROWS = _E // _C
_AW = 128
_NPAD = 10240
_NPW = _NPAD // _NS


def _edge_dense_body(ea_ref, w1_ref, b1_ref, wc_ref, bc_ref,
                     v0_ref, v1_ref, v2_ref, ae_ref):
    ea = ea_ref[...]
    h1 = jnp.maximum(
        jnp.dot(ea, w1_ref[...], preferred_element_type=jnp.float32)
        + b1_ref[...], 0.0)
    t = jnp.dot(h1, wc_ref[...], preferred_element_type=jnp.float32) + bc_ref[...]
    v0_ref[...] = t[:, 0:64]
    v1_ref[...] = t[:, 64:128]
    v2_ref[...] = t[:, 128:192]
    ae_ref[...] = t[:, 192:193]


def _edge_dense(edge_attr, We1, be1, Wc, bc):
    E = edge_attr.shape[0]
    grid = (E // _EBLK,)
    return pl.pallas_call(
        _edge_dense_body,
        grid=grid,
        in_specs=[
            pl.BlockSpec((_EBLK, 16), lambda i: (i, 0)),
            pl.BlockSpec((16, 32), lambda i: (0, 0)),
            pl.BlockSpec((32,), lambda i: (0,)),
            pl.BlockSpec((32, 256), lambda i: (0, 0)),
            pl.BlockSpec((256,), lambda i: (0,)),
        ],
        out_specs=[
            pl.BlockSpec((_EBLK, 64), lambda i: (i, 0)),
            pl.BlockSpec((_EBLK, 64), lambda i: (i, 0)),
            pl.BlockSpec((_EBLK, 64), lambda i: (i, 0)),
            pl.BlockSpec((_EBLK, 1), lambda i: (i, 0)),
        ],
        out_shape=[
            jax.ShapeDtypeStruct((E, 64), jnp.float32),
            jax.ShapeDtypeStruct((E, 64), jnp.float32),
            jax.ShapeDtypeStruct((E, 64), jnp.float32),
            jax.ShapeDtypeStruct((E, 1), jnp.float32),
        ],
    )(edge_attr, We1, be1, Wc, bc)


def _sc_propagate_body(tsrc_h, tdst_h, v_h, ae_h, src_h, dst_h, out_h,
                       idxs_c, idxd_c, ae_c, vv, gs, gd, acc,
                       sem1, sem2, sem3, sem4):
    cid = lax.axis_index("c")
    sid = lax.axis_index("s")
    wid = sid * _NC + cid

    def _zrow(i, carry):
        for k in range(_AW // 16):
            gd[0, i, pl.ds(k * 16, 16)] = jnp.zeros((16,), jnp.float32)
        return carry
    lax.fori_loop(0, _C, _zrow, 0)
    abase = sid * _NPW
    for z in range(_NPW // _C):
        pltpu.sync_copy(gd.at[0].at[pl.ds(0, _C)],
                        acc.at[pl.ds(abase + z * _C, _C)])
    plsc.subcore_barrier()

    crow0 = wid * _NCH

    def _lin(m):
        row = crow0 + m
        return [pltpu.make_async_copy(src_h.at[row],
                                      idxs_c.at[lax.rem(m, 3)], sem1),
                pltpu.make_async_copy(dst_h.at[row],
                                      idxd_c.at[lax.rem(m, 4)], sem1)]

    def _linv(m):
        row = crow0 + m
        return [pltpu.make_async_copy(ae_h.at[row],
                                      ae_c.at[lax.rem(m, 2)], sem4),
                pltpu.make_async_copy(v_h.at[row],
                                      vv.at[lax.rem(m, 2)], sem4)]

    def _gath(m):
        return [pltpu.make_async_copy(tsrc_h.at[idxs_c.at[lax.rem(m, 3)]],
                                      gs.at[lax.rem(m, 2)], sem2),
                pltpu.make_async_copy(tdst_h.at[idxd_c.at[lax.rem(m, 4)]],
                                      gd.at[lax.rem(m, 2)], sem2)]

    def _scat(m):
        return pltpu.make_async_copy(gs.at[lax.rem(m, 2)],
                                     acc.at[idxd_c.at[lax.rem(m, 4)]], sem3)

    for c0 in _lin(0):
        c0.start()
    for c1 in _lin(1):
        c1.start()
    for c2 in _linv(0):
        c2.start()
    for c0 in _lin(0):
        c0.wait()
    for g0 in _gath(0):
        g0.start()

    def _chunk(k, carry):
        @pl.when(k < _NCH - 2)
        def _():
            for c in _lin(k + 2):
                c.start()

        @pl.when(k > 0)
        def _():
            _scat(k - 1).wait()

        @pl.when(k < _NCH - 1)
        def _():
            for c in _lin(k + 1):
                c.wait()
            for g in _gath(k + 1):
                g.start()
            for c in _linv(k + 1):
                c.start()

        for g in _gath(k):
            g.wait()
        for c in _linv(k):
            c.wait()
        j = lax.rem(k, 2)
        x = lax.rem(k, 2)

        @plsc.parallel_loop(0, _C, unroll=4)
        def _edge(c):
            lv = gs[x, c, pl.ds(64, 16)] + gd[x, c, pl.ds(64, 16)] + ae_c[j, c]
            ex = jnp.exp(lv)
            for k2 in range(4):
                sl = pl.ds(k2 * 16, 16)
                r = jnp.maximum(gs[x, c, sl] + gd[x, c, sl] + vv[j, c, sl],
                                0.0)
                gs[x, c, sl] = r * ex
            gs[x, c, pl.ds(64, 16)] = ex
        _scat(k).start(add=True)
        return carry
    lax.fori_loop(0, _NCH, _chunk, 0)
    _scat(_NCH - 1).wait()
    plsc.subcore_barrier()
    pltpu.sync_copy(acc.at[pl.ds(abase, _NPW)],
                    out_h.at[cid, pl.ds(abase, _NPW)])


@functools.partial(
    pl.kernel,
    out_type=jax.ShapeDtypeStruct((_NC, _NPAD, _AW), jnp.float32),
    mesh=plsc.VectorSubcoreMesh(core_axis_name="c", subcore_axis_name="s"),
    scratch_types=[
        pltpu.VMEM((3, _C), jnp.int32),
        pltpu.VMEM((4, _C), jnp.int32),
        pltpu.VMEM((2, _C, 16), jnp.float32),
        pltpu.VMEM((2, _C, 64), jnp.float32),
        pltpu.VMEM((2, _C, _AW), jnp.float32),
        pltpu.VMEM((2, _C, _AW), jnp.float32),
        pltpu.VMEM_SHARED((_NPAD, _AW), jnp.float32),
        pltpu.SemaphoreType.DMA,
        pltpu.SemaphoreType.DMA,
        pltpu.SemaphoreType.DMA,
        pltpu.SemaphoreType.DMA,
    ],
)
def _sc_propagate(tsrc_h, tdst_h, v_h, ae_h, src_h, dst_h, out_h,
                  idxs_c, idxd_c, ae_c, vv, gs, gd, acc,
                  sem1, sem2, sem3, sem4):
    _sc_propagate_body(tsrc_h, tdst_h, v_h, ae_h, src_h, dst_h, out_h,
                       idxs_c, idxd_c, ae_c, vv, gs, gd, acc,
                       sem1, sem2, sem3, sem4)


def kernel(x, edge_index, edge_attr, batch, params):
    p = params
    src2 = edge_index[0].reshape(_ROWS, _C)
    dst2 = edge_index[1].reshape(_ROWS, _C)
    Wa = p['Wa']

    h = x @ p['W_node'] + p['b_node']

    Wc_parts = [p['We2'] @ p['Wc%da' % i][64:128] for i in range(3)]
    bc_parts = [p['be2'] @ p['Wc%da' % i][64:128] for i in range(3)]
    wa_e = p['We2'] @ Wa[128:192]
    ba_e = p['be2'] @ Wa[128:192] + p['ba']
    Wc = jnp.concatenate(
        Wc_parts + [jnp.pad(wa_e, ((0, 0), (0, 63)))], axis=1)
    bc = jnp.concatenate(
        bc_parts + [jnp.pad(ba_e, (0, 63))], axis=0)

    v0, v1, v2, ae = _edge_dense(edge_attr, p['We1'], p['be1'], Wc, bc)
    v = {0: v0.reshape(_ROWS, _C, 64),
         1: v1.reshape(_ROWS, _C, 64),
         2: v2.reshape(_ROWS, _C, 64)}
    a_e = ae[:, 0]
    ae_adj = a_e - jnp.max(a_e)
    ae_b = jnp.broadcast_to(ae_adj.reshape(_ROWS, _C, 1), (_ROWS, _C, 16))

    def propagate(h, ci):
        Wca = p['Wc%da' % ci]
        a_src = h @ Wa[64:128]
        a_dst = h @ Wa[0:64]
        asb = jnp.broadcast_to(a_src - jnp.max(a_src), (_N, 16))
        adb = jnp.broadcast_to(a_dst - jnp.max(a_dst), (_N, 16))
        pad = ((0, _NPAD - _N), (0, 0))
        zp = jnp.zeros((_N, _AW - 80), jnp.float32)
        tsrc = jnp.pad(jnp.concatenate([h @ Wca[0:64], asb, zp], axis=1), pad)
        tdst = jnp.pad(jnp.concatenate(
            [h @ Wca[128:192] + p['bc%da' % ci], adb, zp], axis=1), pad)
        out = _sc_propagate(tsrc, tdst, v[ci], ae_b, src2, dst2)
        accu = (out[0] + out[1])[:_N]
        S = jnp.sum(accu[:, 64])
        return (accu[:, 0:64] @ p['Wc%db' % ci]
                + accu[:, 64:65] * p['bc%db' % ci]) / S

    for i in range(3):
        h = h + jax.nn.relu(propagate(h, i))
    xs = [h]
    for t in range(2):
        h = jax.nn.relu(h @ p['Wt%d' % t] + p['bt%d' % t] + propagate(h, 2))
        xs.append(h)
    h = (xs[0] + xs[1] + xs[2]) * (1.0 / 3.0)
    h = h @ p['Wo'] + p['bo']
    onehot = (batch[:, None] ==
              jnp.arange(_NUM_GRAPHS, dtype=batch.dtype)[None, :]).astype(h.dtype)
    g = onehot.T @ h
    return g @ p['Wf'] + p['bf']


# speedup vs baseline: 1.0182x; 1.0182x over previous
import functools

import jax
import jax.numpy as jnp
from jax import lax
from jax.experimental import pallas as pl
from jax.experimental.pallas import tpu as pltpu
from jax.experimental.pallas import tpu_sc as plsc

_N = 10000
_E = 320000
_HID = 64
_NUM_GRAPHS = 64
_EBLK = 6400

_NC = 2
_NS = 16
_NW = _NC * _NS
_C = 40
_EPT = _E // _NW
_NCH = _EPT // _C
_ROWS = _E // _C
_AW = 128
_NPAD = 10240
_NPW = _NPAD // _NS


def _edge_dense_body(ea_ref, w1_ref, b1_ref, wc_ref, bc_ref,
                     v0_ref, v1_ref, v2_ref, ae_ref):
    ea = ea_ref[...]
    h1 = jnp.maximum(
        jnp.dot(ea, w1_ref[...], preferred_element_type=jnp.float32)
        + b1_ref[...], 0.0)
    t = jnp.dot(h1, wc_ref[...], preferred_element_type=jnp.float32) + bc_ref[...]
    v0_ref[...] = t[:, 0:64]
    v1_ref[...] = t[:, 64:128]
    v2_ref[...] = t[:, 128:192]
    ae_ref[...] = t[:, 192:193]


def _edge_dense(edge_attr, We1, be1, Wc, bc):
    E = edge_attr.shape[0]
    grid = (E // _EBLK,)
    return pl.pallas_call(
        _edge_dense_body,
        grid=grid,
        in_specs=[
            pl.BlockSpec((_EBLK, 16), lambda i: (i, 0)),
            pl.BlockSpec((16, 32), lambda i: (0, 0)),
            pl.BlockSpec((32,), lambda i: (0,)),
            pl.BlockSpec((32, 256), lambda i: (0, 0)),
            pl.BlockSpec((256,), lambda i: (0,)),
        ],
        out_specs=[
            pl.BlockSpec((_EBLK, 64), lambda i: (i, 0)),
            pl.BlockSpec((_EBLK, 64), lambda i: (i, 0)),
            pl.BlockSpec((_EBLK, 64), lambda i: (i, 0)),
            pl.BlockSpec((_EBLK, 1), lambda i: (i, 0)),
        ],
        out_shape=[
            jax.ShapeDtypeStruct((E, 64), jnp.float32),
            jax.ShapeDtypeStruct((E, 64), jnp.float32),
            jax.ShapeDtypeStruct((E, 64), jnp.float32),
            jax.ShapeDtypeStruct((E, 1), jnp.float32),
        ],
    )(edge_attr, We1, be1, Wc, bc)


def _sc_propagate_body(tsrc_h, tdst_h, v_h, ae_h, src_h, dst_h, out_h,
                       idxs_c, idxd_c, ae_c, vv, gs, gd, acc,
                       sem1, sem2, sem3, sem4):
    cid = lax.axis_index("c")
    sid = lax.axis_index("s")
    wid = sid * _NC + cid

    def _zrow(i, carry):
        for k in range(_AW // 16):
            gd[0, i, pl.ds(k * 16, 16)] = jnp.zeros((16,), jnp.float32)
        return carry
    lax.fori_loop(0, _C, _zrow, 0)
    abase = sid * _NPW
    for z in range(_NPW // _C):
        pltpu.sync_copy(gd.at[0].at[pl.ds(0, _C)],
                        acc.at[pl.ds(abase + z * _C, _C)])
    plsc.subcore_barrier()

    crow0 = wid * _NCH

    def _lin(m):
        row = crow0 + m
        return [pltpu.make_async_copy(src_h.at[row],
                                      idxs_c.at[lax.rem(m, 3)], sem1),
                pltpu.make_async_copy(dst_h.at[row],
                                      idxd_c.at[lax.rem(m, 4)], sem1)]

    def _linv(m):
        row = crow0 + m
        return [pltpu.make_async_copy(ae_h.at[row],
                                      ae_c.at[lax.rem(m, 2)], sem4),
                pltpu.make_async_copy(v_h.at[row],
                                      vv.at[lax.rem(m, 2)], sem4)]

    def _gath(m):
        return [pltpu.make_async_copy(tsrc_h.at[idxs_c.at[lax.rem(m, 3)]],
                                      gs.at[lax.rem(m, 2)], sem2),
                pltpu.make_async_copy(tdst_h.at[idxd_c.at[lax.rem(m, 4)]],
                                      gd.at[lax.rem(m, 2)], sem2)]

    def _scat(m):
        return pltpu.make_async_copy(gs.at[lax.rem(m, 2)],
                                     acc.at[idxd_c.at[lax.rem(m, 4)]], sem3)

    for c0 in _lin(0):
        c0.start()
    for c1 in _lin(1):
        c1.start()
    for c2 in _linv(0):
        c2.start()
    for c0 in _lin(0):
        c0.wait()
    for g0 in _gath(0):
        g0.start()

    def _chunk(k, carry):
        @pl.when(k > 0)
        def _():
            _scat(k - 1).wait()

        @pl.when(k < _NCH - 2)
        def _():
            for c in _lin(k + 2):
                c.start()

        @pl.when(k < _NCH - 1)
        def _():
            for c in _lin(k + 1):
                c.wait()
            for g in _gath(k + 1):
                g.start()
            for c in _linv(k + 1):
                c.start()

        for g in _gath(k):
            g.wait()
        for c in _linv(k):
            c.wait()
        j = lax.rem(k, 2)
        x = lax.rem(k, 2)

        @plsc.parallel_loop(0, _C, unroll=4)
        def _edge(c):
            lv = gs[x, c, pl.ds(64, 16)] + gd[x, c, pl.ds(64, 16)] + ae_c[j, c]
            ex = jnp.exp(lv)
            for k2 in range(4):
                sl = pl.ds(k2 * 16, 16)
                r = jnp.maximum(gs[x, c, sl] + gd[x, c, sl] + vv[j, c, sl],
                                0.0)
                gs[x, c, sl] = r * ex
            gs[x, c, pl.ds(64, 16)] = ex
        _scat(k).start(add=True)
        return carry
    lax.fori_loop(0, _NCH, _chunk, 0)
    _scat(_NCH - 1).wait()
    plsc.subcore_barrier()
    pltpu.sync_copy(acc.at[pl.ds(abase, _NPW)],
                    out_h.at[cid, pl.ds(abase, _NPW)])


@functools.partial(
    pl.kernel,
    out_type=jax.ShapeDtypeStruct((_NC, _NPAD, _AW), jnp.float32),
    mesh=plsc.VectorSubcoreMesh(core_axis_name="c", subcore_axis_name="s"),
    scratch_types=[
        pltpu.VMEM((3, _C), jnp.int32),
        pltpu.VMEM((4, _C), jnp.int32),
        pltpu.VMEM((2, _C, 16), jnp.float32),
        pltpu.VMEM((2, _C, 64), jnp.float32),
        pltpu.VMEM((2, _C, _AW), jnp.float32),
        pltpu.VMEM((2, _C, _AW), jnp.float32),
        pltpu.VMEM_SHARED((_NPAD, _AW), jnp.float32),
        pltpu.SemaphoreType.DMA,
        pltpu.SemaphoreType.DMA,
        pltpu.SemaphoreType.DMA,
        pltpu.SemaphoreType.DMA,
    ],
)
def _sc_propagate(tsrc_h, tdst_h, v_h, ae_h, src_h, dst_h, out_h,
                  idxs_c, idxd_c, ae_c, vv, gs, gd, acc,
                  sem1, sem2, sem3, sem4):
    _sc_propagate_body(tsrc_h, tdst_h, v_h, ae_h, src_h, dst_h, out_h,
                       idxs_c, idxd_c, ae_c, vv, gs, gd, acc,
                       sem1, sem2, sem3, sem4)


def kernel(x, edge_index, edge_attr, batch, params):
    p = params
    src2 = edge_index[0].reshape(_ROWS, _C)
    dst2 = edge_index[1].reshape(_ROWS, _C)
    Wa = p['Wa']

    h = x @ p['W_node'] + p['b_node']

    Wc_parts = [p['We2'] @ p['Wc%da' % i][64:128] for i in range(3)]
    bc_parts = [p['be2'] @ p['Wc%da' % i][64:128] for i in range(3)]
    wa_e = p['We2'] @ Wa[128:192]
    ba_e = p['be2'] @ Wa[128:192] + p['ba']
    Wc = jnp.concatenate(
        Wc_parts + [jnp.pad(wa_e, ((0, 0), (0, 63)))], axis=1)
    bc = jnp.concatenate(
        bc_parts + [jnp.pad(ba_e, (0, 63))], axis=0)

    v0, v1, v2, ae = _edge_dense(edge_attr, p['We1'], p['be1'], Wc, bc)
    v = {0: v0.reshape(_ROWS, _C, 64),
         1: v1.reshape(_ROWS, _C, 64),
         2: v2.reshape(_ROWS, _C, 64)}
    a_e = ae[:, 0]
    ae_adj = a_e - jnp.max(a_e)
    ae_b = jnp.broadcast_to(ae_adj.reshape(_ROWS, _C, 1), (_ROWS, _C, 16))

    def propagate(h, ci):
        Wca = p['Wc%da' % ci]
        a_src = h @ Wa[64:128]
        a_dst = h @ Wa[0:64]
        asb = jnp.broadcast_to(a_src - jnp.max(a_src), (_N, 16))
        adb = jnp.broadcast_to(a_dst - jnp.max(a_dst), (_N, 16))
        pad = ((0, _NPAD - _N), (0, 0))
        zp = jnp.zeros((_N, _AW - 80), jnp.float32)
        tsrc = jnp.pad(jnp.concatenate([h @ Wca[0:64], asb, zp], axis=1), pad)
        tdst = jnp.pad(jnp.concatenate(
            [h @ Wca[128:192] + p['bc%da' % ci], adb, zp], axis=1), pad)
        out = _sc_propagate(tsrc, tdst, v[ci], ae_b, src2, dst2)
        accu = (out[0] + out[1])[:_N]
        S = jnp.sum(accu[:, 64])
        return (accu[:, 0:64] @ p['Wc%db' % ci]
                + accu[:, 64:65] * p['bc%db' % ci]) / S

    for i in range(3):
        h = h + jax.nn.relu(propagate(h, i))
    xs = [h]
    for t in range(2):
        h = jax.nn.relu(h @ p['Wt%d' % t] + p['bt%d' % t] + propagate(h, 2))
        xs.append(h)
    h = (xs[0] + xs[1] + xs[2]) * (1.0 / 3.0)
    h = h @ p['Wo'] + p['bo']
    onehot = (batch[:, None] ==
              jnp.arange(_NUM_GRAPHS, dtype=batch.dtype)[None, :]).astype(h.dtype)
    g = onehot.T @ h
    return g @ p['Wf'] + p['bf']
